# Initial kernel scaffold; baseline (speedup 1.0000x reference)
import functools

import jax
import jax.numpy as jnp
from jax import lax
from jax.experimental import pallas as pl
from jax.experimental.pallas import tpu as pltpu
from jax.experimental.pallas import tpu_sc as plsc

N = 10000
E = 320000
D = 128

NC = 2
NS = 16
NW = NC * NS
EPW = E // NW
K = 80
NCHUNK = EPW // K
RPT = N // NS
NPAD = 10240
DSL = NPAD // NS

_mesh = plsc.VectorSubcoreMesh(core_axis_name="c", subcore_axis_name="s")


@functools.partial(
    pl.kernel,
    out_type=jax.ShapeDtypeStruct((NC, NPAD), jnp.float32),
    mesh=_mesh,
    scratch_types=[
        pltpu.VMEM((NCHUNK, K), jnp.int32),
        pltpu.VMEM((K,), jnp.float32),
        pltpu.VMEM((DSL,), jnp.float32),
        pltpu.VMEM_SHARED((NPAD,), jnp.float32),
    ],
)
def _deg_kernel(dst_hbm, ones_hbm, zeros_hbm, deg_out, idx_v, ones_v, zb_v, deg_sh):
    c = lax.axis_index("c")
    s = lax.axis_index("s")
    wid = s * NC + c
    pltpu.sync_copy(zeros_hbm, zb_v)
    pltpu.sync_copy(zb_v, deg_sh.at[pl.ds(s * DSL, DSL)])
    pltpu.sync_copy(ones_hbm, ones_v)
    plsc.subcore_barrier()
    pltpu.sync_copy(dst_hbm.at[wid], idx_v)

    def body(j, carry):
        pltpu.sync_copy(ones_v, deg_sh.at[idx_v.at[j]], add=True)
        return carry

    lax.fori_loop(0, NCHUNK, body, 0)
    plsc.subcore_barrier()
    pltpu.sync_copy(deg_sh.at[pl.ds(s * DSL, DSL)], zb_v)
    pltpu.sync_copy(zb_v, deg_out.at[c].at[pl.ds(s * DSL, DSL)])


RB = 1000


def _norm_body(deg_ref, x_ref, h_ref, norm_ref):
    d = deg_ref[0] + deg_ref[1]
    nrm = lax.rsqrt(jnp.maximum(d, 1.0))
    norm_ref[...] = nrm
    h_ref[...] = x_ref[...] * nrm


_norm_call = pl.pallas_call(
    _norm_body,
    grid=(N // RB,),
    in_specs=[
        pl.BlockSpec((NC, RB, 1), lambda i: (0, i, 0)),
        pl.BlockSpec((RB, D), lambda i: (i, 0)),
    ],
    out_specs=[
        pl.BlockSpec((RB, D), lambda i: (i, 0)),
        pl.BlockSpec((RB, 1), lambda i: (i, 0)),
    ],
    out_shape=[
        jax.ShapeDtypeStruct((N, D), jnp.float32),
        jax.ShapeDtypeStruct((N, 1), jnp.float32),
    ],
)


@functools.partial(
    pl.kernel,
    out_type=jax.ShapeDtypeStruct((NC, N, D), jnp.float32),
    mesh=_mesh,
    scratch_types=[
        pltpu.VMEM((NCHUNK, K), jnp.int32),
        pltpu.VMEM((NCHUNK, K), jnp.int32),
        pltpu.VMEM((K, D), jnp.float32),
        pltpu.VMEM((RPT // 5, D), jnp.float32),
        pltpu.VMEM_SHARED((N, D), jnp.float32),
        pltpu.SemaphoreType.DMA,
    ],
)
def _agg_kernel(h_hbm, src_hbm, dst_hbm, zrows_hbm, agg_out,
                src_v, dst_v, row_v, wb_v, agg_sh, sem):
    c = lax.axis_index("c")
    s = lax.axis_index("s")
    wid = s * NC + c
    pltpu.sync_copy(zrows_hbm, agg_sh.at[pl.ds(s * RPT, RPT)])
    pltpu.sync_copy(src_hbm.at[wid], src_v)
    pltpu.sync_copy(dst_hbm.at[wid], dst_v)
    plsc.subcore_barrier()

    def body(j, carry):
        pltpu.async_copy(h_hbm.at[src_v.at[j]], row_v, sem).wait()
        pltpu.sync_copy(row_v, agg_sh.at[dst_v.at[j]], add=True)
        return carry

    lax.fori_loop(0, NCHUNK, body, 0)
    plsc.subcore_barrier()
    for i in range(5):
        r0 = s * RPT + i * (RPT // 5)
        pltpu.sync_copy(agg_sh.at[pl.ds(r0, RPT // 5)], wb_v)
        pltpu.sync_copy(wb_v, agg_out.at[c].at[pl.ds(r0, RPT // 5)])


def _mm_body(agg_ref, norm_ref, w_ref, b_ref, out_ref):
    a = (agg_ref[0] + agg_ref[1]) * norm_ref[...]
    out_ref[...] = (
        jnp.dot(a, w_ref[...], preferred_element_type=jnp.float32) + b_ref[...]
    )


_mm_call = pl.pallas_call(
    _mm_body,
    grid=(N // RB,),
    in_specs=[
        pl.BlockSpec((NC, RB, D), lambda i: (0, i, 0)),
        pl.BlockSpec((RB, 1), lambda i: (i, 0)),
        pl.BlockSpec((D, D), lambda i: (0, 0)),
        pl.BlockSpec((1, D), lambda i: (0, 0)),
    ],
    out_specs=pl.BlockSpec((RB, D), lambda i: (i, 0)),
    out_shape=jax.ShapeDtypeStruct((N, D), jnp.float32),
)


def kernel(x, edge_index, W, b):
    src = edge_index[0].reshape(NW, NCHUNK, K)
    dst = edge_index[1].reshape(NW, NCHUNK, K)
    ones_k = jnp.ones((K,), jnp.float32)
    zeros_d = jnp.zeros((DSL,), jnp.float32)
    zrows = jnp.zeros((RPT, D), jnp.float32)

    deg_parts = _deg_kernel(dst, ones_k, zeros_d)
    deg = deg_parts[:, :N].reshape(NC, N, 1)
    h, norm = _norm_call(deg, x)
    agg_parts = _agg_kernel(h, src, dst, zrows)
    out = _mm_call(agg_parts, norm, W, b.reshape(1, D))
    return out


# speedup vs baseline: 4.5259x; 4.5259x over previous
import functools

import jax
import jax.numpy as jnp
from jax import lax
from jax.experimental import pallas as pl
from jax.experimental.pallas import tpu as pltpu
from jax.experimental.pallas import tpu_sc as plsc

N = 10000
E = 320000
D = 128

NC = 2
NS = 16
NW = NC * NS
EPW = E // NW
K = 80
NCHUNK = EPW // K
NPAD = 10240
DSL = NPAD // NS

HALF = NPAD // NC
TRASH = HALF
AROWS = HALF + 8
EPT = E // NS
NCH2 = EPT // K
ZR = HALF // NS

_mesh = plsc.VectorSubcoreMesh(core_axis_name="c", subcore_axis_name="s")


@functools.partial(
    pl.kernel,
    out_type=jax.ShapeDtypeStruct((NC * NPAD,), jnp.float32),
    mesh=_mesh,
    scratch_types=[
        pltpu.VMEM((NCHUNK, K), jnp.int32),
        pltpu.VMEM((K,), jnp.float32),
        pltpu.VMEM((DSL,), jnp.float32),
        pltpu.VMEM_SHARED((NPAD,), jnp.float32),
    ],
)
def _deg_kernel(dst_hbm, ones_hbm, zeros_hbm, deg_out, idx_v, ones_v, zb_v, deg_sh):
    c = lax.axis_index("c")
    s = lax.axis_index("s")
    wid = s * NC + c
    pltpu.sync_copy(zeros_hbm, zb_v)
    pltpu.sync_copy(zb_v, deg_sh.at[pl.ds(s * DSL, DSL)])
    pltpu.sync_copy(ones_hbm, ones_v)
    plsc.subcore_barrier()
    pltpu.sync_copy(dst_hbm.at[wid], idx_v)

    def body(j, carry):
        pltpu.sync_copy(ones_v, deg_sh.at[idx_v.at[j]], add=True)
        return carry

    lax.fori_loop(0, NCHUNK, body, 0)
    plsc.subcore_barrier()
    pltpu.sync_copy(deg_sh.at[pl.ds(s * DSL, DSL)], zb_v)
    pltpu.sync_copy(zb_v, deg_out.at[pl.ds(c * NPAD + s * DSL, DSL)])


RB = 1000


def _norm_body(deg_ref, x_ref, h_ref, norm_ref):
    d = deg_ref[0] + deg_ref[1]
    nrm = lax.rsqrt(jnp.maximum(d, 1.0))
    norm_ref[...] = nrm
    h_ref[...] = x_ref[...] * nrm


_norm_call = pl.pallas_call(
    _norm_body,
    grid=(N // RB,),
    in_specs=[
        pl.BlockSpec((NC, RB, 1), lambda i: (0, i, 0)),
        pl.BlockSpec((RB, D), lambda i: (i, 0)),
    ],
    out_specs=[
        pl.BlockSpec((RB, D), lambda i: (i, 0)),
        pl.BlockSpec((RB, 1), lambda i: (i, 0)),
    ],
    out_shape=[
        jax.ShapeDtypeStruct((N, D), jnp.float32),
        jax.ShapeDtypeStruct((N, 1), jnp.float32),
    ],
)


@functools.partial(
    pl.kernel,
    out_type=jax.ShapeDtypeStruct((NC, HALF, D), jnp.float32),
    mesh=_mesh,
    scratch_types=[
        pltpu.VMEM((EPT,), jnp.int32),
        pltpu.VMEM((NCH2, K), jnp.int32),
        pltpu.VMEM((K, D), jnp.float32),
        pltpu.VMEM((ZR // 5, D), jnp.float32),
        pltpu.VMEM_SHARED((AROWS, D), jnp.float32),
        pltpu.SemaphoreType.DMA,
    ],
)
def _agg_kernel(h_hbm, src_hbm, dst_hbm, zrows_hbm, agg_out,
                src_v, dst_v, row_v, wb_v, agg_sh, sem):
    c = lax.axis_index("c")
    s = lax.axis_index("s")
    base = c * HALF
    for i in range(5):
        pltpu.sync_copy(zrows_hbm, agg_sh.at[pl.ds(s * ZR + i * (ZR // 5), ZR // 5)])

    @pl.when(s == 0)
    def _zero_trash():
        pltpu.sync_copy(zrows_hbm.at[pl.ds(0, 8)], agg_sh.at[pl.ds(HALF, 8)])

    pltpu.sync_copy(src_hbm.at[pl.ds(s * EPT, EPT)], src_v)
    pltpu.sync_copy(dst_hbm.at[s], dst_v)

    def remap(j, carry):
        for k in range(K // 16):
            v = dst_v[j, pl.ds(k * 16, 16)]
            loc = v - base
            ok = (loc >= 0) & (loc < HALF)
            dst_v[j, pl.ds(k * 16, 16)] = jnp.where(ok, loc, TRASH)
        return carry

    lax.fori_loop(0, NCH2, remap, 0)
    plsc.subcore_barrier()

    def body(j, carry):
        e0 = pl.multiple_of(j * K, 8)
        pltpu.async_copy(h_hbm.at[src_v.at[pl.ds(e0, K)]], row_v, sem).wait()
        pltpu.sync_copy(row_v, agg_sh.at[dst_v.at[j]], add=True)
        return carry

    lax.fori_loop(0, NCH2, body, 0)
    plsc.subcore_barrier()
    for i in range(5):
        r0 = s * ZR + i * (ZR // 5)
        pltpu.sync_copy(agg_sh.at[pl.ds(r0, ZR // 5)], wb_v)
        pltpu.sync_copy(wb_v, agg_out.at[c].at[pl.ds(r0, ZR // 5)])


def _mm_body(agg_ref, norm_ref, w_ref, b_ref, out_ref):
    a = agg_ref[...] * norm_ref[...]
    out_ref[...] = (
        jnp.dot(a, w_ref[...], preferred_element_type=jnp.float32) + b_ref[...]
    )


_mm_call = pl.pallas_call(
    _mm_body,
    grid=(N // RB,),
    in_specs=[
        pl.BlockSpec((RB, D), lambda i: (i, 0)),
        pl.BlockSpec((RB, 1), lambda i: (i, 0)),
        pl.BlockSpec((D, D), lambda i: (0, 0)),
        pl.BlockSpec((1, D), lambda i: (0, 0)),
    ],
    out_specs=pl.BlockSpec((RB, D), lambda i: (i, 0)),
    out_shape=jax.ShapeDtypeStruct((N, D), jnp.float32),
)


def kernel(x, edge_index, W, b):
    src_w = edge_index[0].reshape(NW, NCHUNK, K)
    dst_w = edge_index[1].reshape(NW, NCHUNK, K)
    src_t = edge_index[0]
    dst_t = edge_index[1].reshape(NS, NCH2, K)
    ones_k = jnp.ones((K,), jnp.float32)
    zeros_d = jnp.zeros((DSL,), jnp.float32)
    zrows = jnp.zeros((ZR // 5, D), jnp.float32)

    deg_parts = _deg_kernel(dst_w, ones_k, zeros_d)
    deg = deg_parts.reshape(NC, NPAD)[:, :N].reshape(NC, N, 1)
    h, norm = _norm_call(deg, x)
    agg_parts = _agg_kernel(h, src_t, dst_t, zrows)
    agg = agg_parts.reshape(NC * HALF, D)[:N]
    out = _mm_call(agg, norm, W, b.reshape(1, D))
    return out
